# baseline (speedup 1.0000x reference)
import functools

import jax
import jax.numpy as jnp
from jax import lax
from jax.experimental import pallas as pl
from jax.experimental.pallas import tpu as pltpu, tpu_sc as plsc

N = 320000
D = 128
B = 2048
D_ADDUCT = 16

NBLK = N // 128
NW = 32
PER_W = NBLK // NW
EXTRA = NBLK - PER_W * NW
PAIRS = PER_W // 2


def _sc_body(x_hbm, ids_hbm, z128_hbm,
             psum_hbm, pfirst_hbm,
             sums_sp, first_sp,
             xbuf, idbuf, firstbuf, prevbuf, idx16,
             semx0, semx1, semi0, semi1, sems0, sems1):
    cid = lax.axis_index("c")
    sid = lax.axis_index("s")
    wid = sid * 2 + cid
    i32 = jnp.int32
    iota = lax.iota(i32, 16)

    pltpu.sync_copy(z128_hbm, sums_sp.at[pl.ds(sid * 128, 128)])

    @pl.when(sid == 0)
    def _():
        pltpu.sync_copy(z128_hbm.at[pl.ds(0, 16)], first_sp)

    pltpu.sync_copy(z128_hbm.at[pl.ds(0, 16)], firstbuf)
    idx16[...] = iota
    plsc.subcore_barrier()

    start = wid * PER_W + jnp.minimum(wid, EXTRA)

    pltpu.sync_copy(ids_hbm.at[jnp.maximum(start - 1, 0)], prevbuf)
    pv = plsc.load_gather(prevbuf, [jnp.full((16,), 127, i32)])
    prev0 = jnp.where(wid == 0, jnp.full((16,), -1, i32), pv)

    def fill(buf, sx, si, blk):
        pltpu.async_copy(x_hbm.at[pl.ds(blk * 128, 128)], xbuf.at[buf], sx)
        pltpu.async_copy(ids_hbm.at[blk], idbuf.at[buf], si)

    def wait_fill(buf, sx, si, blk):
        pltpu.make_async_copy(x_hbm.at[pl.ds(blk * 128, 128)],
                              xbuf.at[buf], sx).wait()
        pltpu.make_async_copy(ids_hbm.at[blk], idbuf.at[buf], si).wait()

    def firsts(buf, blk, prev_last):
        bvec = jnp.full((16,), buf, i32)
        new_last = plsc.load_gather(idbuf, [bvec, jnp.full((16,), 127, i32)])
        for v in range(8):
            p = iota + v * 16
            ids16 = idbuf[buf, pl.ds(v * 16, 16)]
            prv = plsc.load_gather(idbuf, [bvec, jnp.maximum(p - 1, 0)])
            pr = jnp.where(p == 0, prev_last, prv)
            first_m = ids16 != pr
            gpos1 = (blk * 128 + p + 1).astype(jnp.float32)
            hi = lax.shift_right_logical(ids16, 7)
            lo = lax.bitwise_and(ids16, 127)
            plsc.store_scatter(firstbuf, [hi, lo], gpos1, mask=first_m)
        return new_last

    fill(0, semx0, semi0, start)

    def pair(k, prev_last):
        b0 = start + 2 * k
        b1 = b0 + 1
        wait_fill(0, semx0, semi0, b0)
        fill(1, semx1, semi1, b1)
        ds0 = pltpu.async_copy(xbuf.at[0], sums_sp.at[idbuf.at[0]],
                               sems0, add=True)
        pl0 = firsts(0, b0, prev_last)
        wait_fill(1, semx1, semi1, b1)
        ds1 = pltpu.async_copy(xbuf.at[1], sums_sp.at[idbuf.at[1]],
                               sems1, add=True)
        pl1 = firsts(1, b1, pl0)
        ds0.wait()

        @pl.when(k + 1 < PAIRS)
        def _():
            fill(0, semx0, semi0, b0 + 2)

        ds1.wait()
        return pl1

    prev_last = lax.fori_loop(0, PAIRS, pair, prev0)

    @pl.when(wid < EXTRA)
    def _():
        blk = start + PER_W
        fill(0, semx0, semi0, blk)
        wait_fill(0, semx0, semi0, blk)
        pltpu.sync_copy(xbuf.at[0], sums_sp.at[idbuf.at[0]], add=True)
        firsts(0, blk, prev_last)

    pltpu.sync_copy(firstbuf, first_sp.at[idx16], add=True)
    plsc.subcore_barrier()

    out_row = cid * B + sid * 128
    pltpu.sync_copy(sums_sp.at[pl.ds(sid * 128, 128)],
                    psum_hbm.at[pl.ds(out_row, 128)])

    @pl.when(sid == 0)
    def _():
        pltpu.sync_copy(first_sp, pfirst_hbm.at[cid])


_sc_call = functools.partial(
    pl.kernel,
    out_type=(
        jax.ShapeDtypeStruct((2 * B, D), jnp.float32),
        jax.ShapeDtypeStruct((2, 16, 128), jnp.float32),
    ),
    mesh=plsc.VectorSubcoreMesh(core_axis_name="c", subcore_axis_name="s"),
    compiler_params=pltpu.CompilerParams(needs_layout_passes=False),
    scratch_types=[
        pltpu.VMEM_SHARED((B, D), jnp.float32),
        pltpu.VMEM_SHARED((16, 128), jnp.float32),
        pltpu.VMEM((2, 128, D), jnp.float32),
        pltpu.VMEM((2, 128), jnp.int32),
        pltpu.VMEM((16, 128), jnp.float32),
        pltpu.VMEM((128,), jnp.int32),
        pltpu.VMEM((16,), jnp.int32),
        pltpu.SemaphoreType.DMA,
        pltpu.SemaphoreType.DMA,
        pltpu.SemaphoreType.DMA,
        pltpu.SemaphoreType.DMA,
        pltpu.SemaphoreType.DMA,
        pltpu.SemaphoreType.DMA,
    ],
)(_sc_body)

_BIG = 1.0e9


def _combine_body(ps_ref, f0_ref, f1_ref, ad_ref, o_ref):
    s = ps_ref[0:B, :] + ps_ref[B:2 * B, :]
    fs = f0_ref[...] + f1_ref[...]
    present = fs > 0.5
    first = fs - 1.0
    g = jnp.where(present, first, _BIG)
    m = g
    k = 1
    while k < B:
        pad = jnp.full((k, 1), _BIG, jnp.float32)
        m = jnp.minimum(m, jnp.concatenate([m[k:], pad], axis=0))
        k *= 2
    nxt = jnp.concatenate([m[1:], jnp.full((1, 1), _BIG, jnp.float32)],
                          axis=0)
    cnt = jnp.where(present, jnp.minimum(nxt, float(N)) - first, 0.0)
    o_ref[...] = jnp.concatenate([s / jnp.maximum(cnt, 1.0), ad_ref[...]],
                                 axis=1)


_combine = pl.pallas_call(
    _combine_body,
    out_shape=jax.ShapeDtypeStruct((B, D + D_ADDUCT), jnp.float32),
)


def kernel(x, segment_ids, adduct):
    ids2 = segment_ids.reshape(NBLK, 128)
    z128 = jnp.zeros((128, D), jnp.float32)
    psums, pfirst = _sc_call(x, ids2, z128)
    f0 = pfirst[0].reshape(B, 1)
    f1 = pfirst[1].reshape(B, 1)
    return _combine(psums, f0, f1, adduct.astype(jnp.float32))
